# baseline (device time: 52258 ns/iter reference)
import jax
import jax.numpy as jnp
from jax import lax
from jax.experimental import pallas as pl
from jax.experimental.pallas import tpu as pltpu

N_DEV = 4
RAILS = 2


def kernel(A, B):
    m, k = A.shape
    _, n = B.shape
    chunk = m // N_DEV
    half = n // 2
    rail = half // RAILS

    def body(a_ref, b_ref, out_ref, comm_r, comm_l, a_vm, b_vm, b_bf,
             send_r, recv_r, send_l, recv_l, dma_sems):
        my = lax.axis_index("i")
        right = lax.rem(my + 1, N_DEV)
        left = lax.rem(my + N_DEV - 1, N_DEV)

        c_r0 = lax.rem(my + N_DEV - 1, N_DEV)
        c_l0 = lax.rem(my + 1, N_DEV)
        c_h0 = lax.rem(my + 2, N_DEV)
        c_h2 = my

        a_dmas = []
        for idx, c in enumerate((c_r0, c_l0, c_h0, c_h2)):
            d = pltpu.make_async_copy(
                a_ref.at[pl.ds(c * chunk, chunk), :],
                a_vm.at[pl.ds(c * chunk, chunk), :],
                dma_sems.at[idx],
            )
            d.start()
            a_dmas.append(d)
        b_cols = []
        for r in range(RAILS):
            b_cols.append(r * rail)
            b_cols.append(half + r * rail)
        b_dmas = []
        for idx, c0 in enumerate(b_cols):
            d = pltpu.make_async_copy(
                b_ref.at[:, pl.ds(c0, rail)],
                b_vm.at[:, pl.ds(c0, rail)],
                dma_sems.at[4 + idx],
            )
            d.start()
            b_dmas.append(d)

        def a_bf(c):
            return a_vm[pl.ds(c * chunk, chunk), :].astype(jnp.bfloat16)

        def dot_half(c, lo):
            return jax.lax.dot_general(
                a_bf(c), b_bf[:, lo:lo + half],
                (((1,), (0,)), ((), ())),
                preferred_element_type=jnp.float32,
            )

        def make(comm, sems_s, sems_r, r, s, dst):
            return pltpu.make_async_remote_copy(
                src_ref=comm.at[r, s],
                dst_ref=comm.at[r, s + 1],
                send_sem=sems_s.at[r, s],
                recv_sem=sems_r.at[r, s],
                device_id=(dst,),
                device_id_type=pl.DeviceIdType.MESH,
            )

        a_dmas[0].wait()
        a_r = a_bf(c_r0)
        a_dmas[1].wait()
        a_l = a_bf(c_l0)
        pend_r = [None] * RAILS
        pend_l = [None] * RAILS
        first = True
        for r in range(RAILS):
            for i, (comm, pend, sems_s, sems_r, dst, a, lo) in enumerate((
                (comm_r, pend_r, send_r, recv_r, right, a_r, 0),
                (comm_l, pend_l, send_l, recv_l, left, a_l, half),
            )):
                c0 = lo + r * rail
                b_dmas[2 * r + i].wait()
                b_bf[:, c0:c0 + rail] = b_vm[:, c0:c0 + rail].astype(
                    jnp.bfloat16)
                p = jax.lax.dot_general(
                    a, b_bf[:, c0:c0 + rail],
                    (((1,), (0,)), ((), ())),
                    preferred_element_type=jnp.float32,
                )
                comm[r, 0, :, :] = p.astype(comm.dtype)
                if first:
                    barrier = pltpu.get_barrier_semaphore()
                    for nbr in (left, right):
                        pl.semaphore_signal(
                            barrier, inc=1, device_id=(nbr,),
                            device_id_type=pl.DeviceIdType.MESH)
                    pl.semaphore_wait(barrier, 2)
                    first = False
                d = make(comm, sems_s, sems_r, r, 0, dst)
                d.start()
                pend[r] = d

        for s in range(N_DEV - 1):
            if s == 0:
                a_dmas[2].wait()
                p_r = dot_half(c_h0, 0)
                p_l = dot_half(c_h0, half)
            elif s == 1:
                p_r = dot_half(c_l0, 0)
                p_l = dot_half(c_r0, half)
            else:
                a_dmas[3].wait()
                p_r = dot_half(c_h2, 0)
                p_l = dot_half(c_h2, half)
            for r in range(RAILS):
                for comm, pend, sems_s, sems_r, dst, p, lo in (
                    (comm_r, pend_r, send_r, recv_r, right, p_r, 0),
                    (comm_l, pend_l, send_l, recv_l, left, p_l, half),
                ):
                    pend[r].wait()
                    acc = p[:, r * rail:(r + 1) * rail]
                    if s < N_DEV - 2:
                        comm[r, s + 1, :, :] = (
                            acc + comm[r, s + 1, :, :].astype(jnp.float32)
                        ).astype(comm.dtype)
                        d = make(comm, sems_s, sems_r, r, s + 1, dst)
                        d.start()
                        pend[r] = d
                    else:
                        c0 = lo + r * rail
                        out_ref[:, c0:c0 + rail] = (
                            acc + comm[r, s + 1, :, :].astype(jnp.float32))

    return pl.pallas_call(
        body,
        out_shape=jax.ShapeDtypeStruct((chunk, n), jnp.float32),
        in_specs=[
            pl.BlockSpec(memory_space=pl.ANY),
            pl.BlockSpec(memory_space=pl.ANY),
        ],
        out_specs=pl.BlockSpec(memory_space=pltpu.VMEM),
        scratch_shapes=[
            pltpu.VMEM((RAILS, N_DEV, chunk, rail), jnp.bfloat16),
            pltpu.VMEM((RAILS, N_DEV, chunk, rail), jnp.bfloat16),
            pltpu.VMEM((m, k), jnp.float32),
            pltpu.VMEM((k, n), jnp.float32),
            pltpu.VMEM((k, n), jnp.bfloat16),
            pltpu.SemaphoreType.DMA((RAILS, N_DEV - 1)),
            pltpu.SemaphoreType.DMA((RAILS, N_DEV - 1)),
            pltpu.SemaphoreType.DMA((RAILS, N_DEV - 1)),
            pltpu.SemaphoreType.DMA((RAILS, N_DEV - 1)),
            pltpu.SemaphoreType.DMA((4 + 2 * RAILS,)),
        ],
        compiler_params=pltpu.CompilerParams(collective_id=0),
    )(A, B)


# device time: 49413 ns/iter; 1.0576x vs baseline; 1.0576x over previous
import jax
import jax.numpy as jnp
from jax import lax
from jax.experimental import pallas as pl
from jax.experimental.pallas import tpu as pltpu

N_DEV = 4
RAILS = 2


def kernel(A, B):
    m, _ = A.shape
    _, n = B.shape
    chunk = m // N_DEV
    half = n // 2
    rail = half // RAILS

    def body(a_ref, b_ref, out_ref, comm_r, comm_l, b_bf,
             send_r, recv_r, send_l, recv_l):
        my = lax.axis_index("i")
        right = lax.rem(my + 1, N_DEV)
        left = lax.rem(my + N_DEV - 1, N_DEV)

        def a_slice(c):
            return a_ref[pl.ds(c * chunk, chunk), :].astype(jnp.bfloat16)

        def dot_cols(a, lo, w):
            return jax.lax.dot_general(
                a, b_bf[:, lo:lo + w],
                (((1,), (0,)), ((), ())),
                preferred_element_type=jnp.float32,
            )

        def make(comm, sems_s, sems_r, r, s, dst):
            return pltpu.make_async_remote_copy(
                src_ref=comm.at[r, s],
                dst_ref=comm.at[r, s + 1],
                send_sem=sems_s.at[r, s],
                recv_sem=sems_r.at[r, s],
                device_id=(dst,),
                device_id_type=pl.DeviceIdType.MESH,
            )

        a_r = a_slice(lax.rem(my + N_DEV - 1, N_DEV))
        a_l = a_slice(lax.rem(my + 1, N_DEV))
        pend_r = [None] * RAILS
        pend_l = [None] * RAILS
        first = True
        for r in range(RAILS):
            for comm, pend, sems_s, sems_r, dst, a, lo in (
                (comm_r, pend_r, send_r, recv_r, right, a_r, 0),
                (comm_l, pend_l, send_l, recv_l, left, a_l, half),
            ):
                c0 = lo + r * rail
                b_bf[:, c0:c0 + rail] = b_ref[:, c0:c0 + rail].astype(
                    jnp.bfloat16)
                p = dot_cols(a, c0, rail)
                comm[r, 0, :, :] = p.astype(comm.dtype)
                if first:
                    barrier = pltpu.get_barrier_semaphore()
                    for nbr in (left, right):
                        pl.semaphore_signal(
                            barrier, inc=1, device_id=(nbr,),
                            device_id_type=pl.DeviceIdType.MESH)
                    pl.semaphore_wait(barrier, 2)
                    first = False
                d = make(comm, sems_s, sems_r, r, 0, dst)
                d.start()
                pend[r] = d

        for s in range(N_DEV - 1):
            if s == 0:
                a = a_slice(lax.rem(my + 2, N_DEV))
                p_r = dot_cols(a, 0, half)
                p_l = dot_cols(a, half, half)
            elif s == 1:
                p_r = dot_cols(a_l, 0, half)
                p_l = dot_cols(a_r, half, half)
            else:
                a = a_slice(my)
                p_r = dot_cols(a, 0, half)
                p_l = dot_cols(a, half, half)
            for r in range(RAILS):
                for comm, pend, sems_s, sems_r, dst, p, lo in (
                    (comm_r, pend_r, send_r, recv_r, right, p_r, 0),
                    (comm_l, pend_l, send_l, recv_l, left, p_l, half),
                ):
                    pend[r].wait()
                    acc = p[:, r * rail:(r + 1) * rail]
                    if s < N_DEV - 2:
                        comm[r, s + 1, :, :] = (
                            acc + comm[r, s + 1, :, :].astype(jnp.float32)
                        ).astype(comm.dtype)
                        d = make(comm, sems_s, sems_r, r, s + 1, dst)
                        d.start()
                        pend[r] = d
                    else:
                        c0 = lo + r * rail
                        out_ref[:, c0:c0 + rail] = (
                            acc + comm[r, s + 1, :, :].astype(jnp.float32))

    return pl.pallas_call(
        body,
        out_shape=jax.ShapeDtypeStruct((chunk, n), jnp.float32),
        in_specs=[
            pl.BlockSpec(memory_space=pltpu.VMEM),
            pl.BlockSpec(memory_space=pltpu.VMEM),
        ],
        out_specs=pl.BlockSpec(memory_space=pltpu.VMEM),
        scratch_shapes=[
            pltpu.VMEM((RAILS, N_DEV, chunk, rail), jnp.bfloat16),
            pltpu.VMEM((RAILS, N_DEV, chunk, rail), jnp.bfloat16),
            pltpu.VMEM((A.shape[1], n), jnp.bfloat16),
            pltpu.SemaphoreType.DMA((RAILS, N_DEV - 1)),
            pltpu.SemaphoreType.DMA((RAILS, N_DEV - 1)),
            pltpu.SemaphoreType.DMA((RAILS, N_DEV - 1)),
            pltpu.SemaphoreType.DMA((RAILS, N_DEV - 1)),
        ],
        compiler_params=pltpu.CompilerParams(collective_id=0),
    )(A, B)


# device time: 46304 ns/iter; 1.1286x vs baseline; 1.0671x over previous
import jax
import jax.numpy as jnp
from jax import lax
from jax.experimental import pallas as pl
from jax.experimental.pallas import tpu as pltpu

N_DEV = 4
RAILS = 2


def kernel(A, B):
    m, _ = A.shape
    _, n = B.shape
    chunk = m // N_DEV
    half = n // 2
    rail = half // RAILS

    def body(a_ref, b_ref, out_ref, comm_r, comm_l, a_vm, b_vm, b_bf,
             send_r, recv_r, send_l, recv_l, dma_sems):
        my = lax.axis_index("i")
        right = lax.rem(my + 1, N_DEV)
        left = lax.rem(my + N_DEV - 1, N_DEV)

        c_r0 = lax.rem(my + N_DEV - 1, N_DEV)
        c_l0 = lax.rem(my + 1, N_DEV)

        def a_dma(c, sem_idx):
            return pltpu.make_async_copy(
                a_ref.at[pl.ds(c * chunk, chunk), :],
                a_vm.at[pl.ds(c * chunk, chunk), :],
                dma_sems.at[sem_idx],
            )

        def b_dma(c0, sem_idx):
            return pltpu.make_async_copy(
                b_ref.at[:, pl.ds(c0, rail)],
                b_vm.at[:, pl.ds(c0, rail)],
                dma_sems.at[sem_idx],
            )

        dmas = [
            b_dma(0, 0),
            a_dma(c_r0, 1),
            b_dma(half, 2),
            a_dma(c_l0, 3),
        ]
        for r in range(1, RAILS):
            dmas.append(b_dma(r * rail, 2 + 2 * r))
            dmas.append(b_dma(half + r * rail, 3 + 2 * r))
        base = 2 + 2 * RAILS
        dmas.append(a_dma(lax.rem(my + 2, N_DEV), base))
        dmas.append(a_dma(my, base + 1))
        for d in dmas:
            d.start()

        def a_slice(c):
            return a_vm[pl.ds(c * chunk, chunk), :].astype(jnp.bfloat16)

        def dot_cols(a, lo, w):
            return jax.lax.dot_general(
                a, b_bf[:, lo:lo + w],
                (((1,), (0,)), ((), ())),
                preferred_element_type=jnp.float32,
            )

        def make(comm, sems_s, sems_r, r, s, dst):
            return pltpu.make_async_remote_copy(
                src_ref=comm.at[r, s],
                dst_ref=comm.at[r, s + 1],
                send_sem=sems_s.at[r, s],
                recv_sem=sems_r.at[r, s],
                device_id=(dst,),
                device_id_type=pl.DeviceIdType.MESH,
            )

        dmas[1].wait()
        a_r = a_slice(c_r0)
        dmas[3].wait()
        a_l = a_slice(c_l0)
        pend_r = [None] * RAILS
        pend_l = [None] * RAILS
        first = True
        for r in range(RAILS):
            for i, (comm, pend, sems_s, sems_r, dst, a, lo) in enumerate((
                (comm_r, pend_r, send_r, recv_r, right, a_r, 0),
                (comm_l, pend_l, send_l, recv_l, left, a_l, half),
            )):
                c0 = lo + r * rail
                dmas[2 * i if r == 0 else 2 + 2 * r + i].wait()
                b_bf[:, c0:c0 + rail] = b_vm[:, c0:c0 + rail].astype(
                    jnp.bfloat16)
                p = dot_cols(a, c0, rail)
                comm[r, 0, :, :] = p.astype(comm.dtype)
                if first:
                    barrier = pltpu.get_barrier_semaphore()
                    for nbr in (left, right):
                        pl.semaphore_signal(
                            barrier, inc=1, device_id=(nbr,),
                            device_id_type=pl.DeviceIdType.MESH)
                    pl.semaphore_wait(barrier, 2)
                    first = False
                d = make(comm, sems_s, sems_r, r, 0, dst)
                d.start()
                pend[r] = d

        for s in range(N_DEV - 1):
            if s == 0:
                dmas[base].wait()
                a = a_slice(lax.rem(my + 2, N_DEV))
                p_r = dot_cols(a, 0, half)
                p_l = dot_cols(a, half, half)
            elif s == 1:
                p_r = dot_cols(a_l, 0, half)
                p_l = dot_cols(a_r, half, half)
            else:
                dmas[base + 1].wait()
                a = a_slice(my)
                p_r = dot_cols(a, 0, half)
                p_l = dot_cols(a, half, half)
            for r in range(RAILS):
                for comm, pend, sems_s, sems_r, dst, p, lo in (
                    (comm_r, pend_r, send_r, recv_r, right, p_r, 0),
                    (comm_l, pend_l, send_l, recv_l, left, p_l, half),
                ):
                    pend[r].wait()
                    acc = p[:, r * rail:(r + 1) * rail]
                    if s < N_DEV - 2:
                        comm[r, s + 1, :, :] = (
                            acc + comm[r, s + 1, :, :].astype(jnp.float32)
                        ).astype(comm.dtype)
                        d = make(comm, sems_s, sems_r, r, s + 1, dst)
                        d.start()
                        pend[r] = d
                    else:
                        c0 = lo + r * rail
                        out_ref[:, c0:c0 + rail] = (
                            acc + comm[r, s + 1, :, :].astype(jnp.float32))

    return pl.pallas_call(
        body,
        out_shape=jax.ShapeDtypeStruct((chunk, n), jnp.float32),
        in_specs=[
            pl.BlockSpec(memory_space=pl.ANY),
            pl.BlockSpec(memory_space=pl.ANY),
        ],
        out_specs=pl.BlockSpec(memory_space=pltpu.VMEM),
        scratch_shapes=[
            pltpu.VMEM((RAILS, N_DEV, chunk, rail), jnp.bfloat16),
            pltpu.VMEM((RAILS, N_DEV, chunk, rail), jnp.bfloat16),
            pltpu.VMEM((m, A.shape[1]), jnp.float32),
            pltpu.VMEM((A.shape[1], n), jnp.float32),
            pltpu.VMEM((A.shape[1], n), jnp.bfloat16),
            pltpu.SemaphoreType.DMA((RAILS, N_DEV - 1)),
            pltpu.SemaphoreType.DMA((RAILS, N_DEV - 1)),
            pltpu.SemaphoreType.DMA((RAILS, N_DEV - 1)),
            pltpu.SemaphoreType.DMA((RAILS, N_DEV - 1)),
            pltpu.SemaphoreType.DMA((4 + 2 * RAILS,)),
        ],
        compiler_params=pltpu.CompilerParams(
            collective_id=0, vmem_limit_bytes=48 * 1024 * 1024),
    )(A, B)


# device time: 45741 ns/iter; 1.1425x vs baseline; 1.0123x over previous
import jax
import jax.numpy as jnp
from jax import lax
from jax.experimental import pallas as pl
from jax.experimental.pallas import tpu as pltpu

N_DEV = 4
RAILS = 4


def kernel(A, B):
    m, _ = A.shape
    _, n = B.shape
    chunk = m // N_DEV
    half = n // 2
    rail = half // RAILS

    def body(a_ref, b_ref, out_ref, comm_r, comm_l, a_vm, b_vm, b_bf,
             send_r, recv_r, send_l, recv_l, dma_sems):
        my = lax.axis_index("i")
        right = lax.rem(my + 1, N_DEV)
        left = lax.rem(my + N_DEV - 1, N_DEV)

        c_r0 = lax.rem(my + N_DEV - 1, N_DEV)
        c_l0 = lax.rem(my + 1, N_DEV)

        def a_dma(c, sem_idx):
            return pltpu.make_async_copy(
                a_ref.at[pl.ds(c * chunk, chunk), :],
                a_vm.at[pl.ds(c * chunk, chunk), :],
                dma_sems.at[sem_idx],
            )

        def b_dma(c0, sem_idx):
            return pltpu.make_async_copy(
                b_ref.at[:, pl.ds(c0, rail)],
                b_vm.at[:, pl.ds(c0, rail)],
                dma_sems.at[sem_idx],
            )

        dmas = [
            b_dma(0, 0),
            a_dma(c_r0, 1),
            b_dma(half, 2),
            a_dma(c_l0, 3),
        ]
        for r in range(1, RAILS):
            dmas.append(b_dma(r * rail, 2 + 2 * r))
            dmas.append(b_dma(half + r * rail, 3 + 2 * r))
        base = 2 + 2 * RAILS
        dmas.append(a_dma(lax.rem(my + 2, N_DEV), base))
        dmas.append(a_dma(my, base + 1))
        for d in dmas:
            d.start()

        def a_slice(c):
            return a_vm[pl.ds(c * chunk, chunk), :].astype(jnp.bfloat16)

        def dot_cols(a, lo, w):
            return jax.lax.dot_general(
                a, b_bf[:, lo:lo + w],
                (((1,), (0,)), ((), ())),
                preferred_element_type=jnp.float32,
            )

        def make(comm, sems_s, sems_r, r, s, dst):
            return pltpu.make_async_remote_copy(
                src_ref=comm.at[r, s],
                dst_ref=comm.at[r, s + 1],
                send_sem=sems_s.at[r, s],
                recv_sem=sems_r.at[r, s],
                device_id=(dst,),
                device_id_type=pl.DeviceIdType.MESH,
            )

        dmas[1].wait()
        a_r = a_slice(c_r0)
        dmas[3].wait()
        a_l = a_slice(c_l0)
        pend_r = [None] * RAILS
        pend_l = [None] * RAILS
        first = True
        for r in range(RAILS):
            for i, (comm, pend, sems_s, sems_r, dst, a, lo) in enumerate((
                (comm_r, pend_r, send_r, recv_r, right, a_r, 0),
                (comm_l, pend_l, send_l, recv_l, left, a_l, half),
            )):
                c0 = lo + r * rail
                dmas[2 * i if r == 0 else 2 + 2 * r + i].wait()
                b_bf[:, c0:c0 + rail] = b_vm[:, c0:c0 + rail].astype(
                    jnp.bfloat16)
                p = dot_cols(a, c0, rail)
                comm[r, 0, :, :] = p.astype(comm.dtype)
                if first:
                    barrier = pltpu.get_barrier_semaphore()
                    for nbr in (left, right):
                        pl.semaphore_signal(
                            barrier, inc=1, device_id=(nbr,),
                            device_id_type=pl.DeviceIdType.MESH)
                    pl.semaphore_wait(barrier, 2)
                    first = False
                d = make(comm, sems_s, sems_r, r, 0, dst)
                d.start()
                pend[r] = d

        for s in range(N_DEV - 1):
            if s == 0:
                dmas[base].wait()
                a = a_slice(lax.rem(my + 2, N_DEV))
                p_r = dot_cols(a, 0, half)
                p_l = dot_cols(a, half, half)
            elif s == 1:
                p_r = dot_cols(a_l, 0, half)
                p_l = dot_cols(a_r, half, half)
            else:
                dmas[base + 1].wait()
                a = a_slice(my)
                p_r = dot_cols(a, 0, half)
                p_l = dot_cols(a, half, half)
            for r in range(RAILS):
                for comm, pend, sems_s, sems_r, dst, p, lo in (
                    (comm_r, pend_r, send_r, recv_r, right, p_r, 0),
                    (comm_l, pend_l, send_l, recv_l, left, p_l, half),
                ):
                    pend[r].wait()
                    acc = p[:, r * rail:(r + 1) * rail]
                    if s < N_DEV - 2:
                        comm[r, s + 1, :, :] = (
                            acc + comm[r, s + 1, :, :].astype(jnp.float32)
                        ).astype(comm.dtype)
                        d = make(comm, sems_s, sems_r, r, s + 1, dst)
                        d.start()
                        pend[r] = d
                    else:
                        c0 = lo + r * rail
                        out_ref[:, c0:c0 + rail] = (
                            acc + comm[r, s + 1, :, :].astype(jnp.float32))

    return pl.pallas_call(
        body,
        out_shape=jax.ShapeDtypeStruct((chunk, n), jnp.float32),
        in_specs=[
            pl.BlockSpec(memory_space=pl.ANY),
            pl.BlockSpec(memory_space=pl.ANY),
        ],
        out_specs=pl.BlockSpec(memory_space=pltpu.VMEM),
        scratch_shapes=[
            pltpu.VMEM((RAILS, N_DEV, chunk, rail), jnp.bfloat16),
            pltpu.VMEM((RAILS, N_DEV, chunk, rail), jnp.bfloat16),
            pltpu.VMEM((m, A.shape[1]), jnp.float32),
            pltpu.VMEM((A.shape[1], n), jnp.float32),
            pltpu.VMEM((A.shape[1], n), jnp.bfloat16),
            pltpu.SemaphoreType.DMA((RAILS, N_DEV - 1)),
            pltpu.SemaphoreType.DMA((RAILS, N_DEV - 1)),
            pltpu.SemaphoreType.DMA((RAILS, N_DEV - 1)),
            pltpu.SemaphoreType.DMA((RAILS, N_DEV - 1)),
            pltpu.SemaphoreType.DMA((4 + 2 * RAILS,)),
        ],
        compiler_params=pltpu.CompilerParams(
            collective_id=0, vmem_limit_bytes=48 * 1024 * 1024),
    )(A, B)
